# baseline (speedup 1.0000x reference)
import functools

import jax
import jax.numpy as jnp
from jax import lax
from jax.experimental import pallas as pl
from jax.experimental.pallas import tpu as pltpu
from jax.experimental.pallas import tpu_sc as plsc

N_COLS = 8192
N_ROWS = 4096
L = 16
NC, NS = 2, 16
NW = NC * NS
ROWS_W = N_ROWS // NW
CH = 1
VECS = N_COLS // L
ACCS = 4
NEG = -1.0


def _bcast(scalar):
    return jnp.full((L,), scalar, dtype=jnp.float32)


def _tri_update(tri, x):
    t1, t2, t3 = tri
    n1 = jnp.maximum(t1, x)
    lo = jnp.minimum(t1, x)
    n2 = jnp.maximum(t2, lo)
    lo2 = jnp.minimum(t2, lo)
    n3 = jnp.maximum(t3, lo2)
    return n1, n2, n3


def _tri_merge(a, b):
    a1, a2, a3 = a
    b1, b2, b3 = b
    c1 = jnp.maximum(a1, b1)
    c2 = jnp.maximum(jnp.maximum(a2, b2), jnp.minimum(a1, b1))
    c3 = jnp.maximum(
        jnp.maximum(a3, b3),
        jnp.maximum(jnp.minimum(a2, b1), jnp.minimum(a1, b2)),
    )
    return c1, c2, c3


def _row_stats(buf, r):
    neg = _bcast(NEG)
    init = tuple((neg, neg, neg) for _ in range(ACCS))

    @plsc.parallel_loop(0, VECS, ACCS, unroll=4, carry=init)
    def tris(i, tris):
        return tuple(
            _tri_update(tris[j], buf[r, pl.ds((i + j) * L, L)])
            for j in range(ACCS)
        )

    t = tris[0]
    for j in range(1, ACCS):
        t = _tri_merge(t, tris[j])
    t1, t2, t3 = t

    one = _bcast(1.0)
    zero = _bcast(0.0)
    negv = _bcast(NEG)
    three = _bcast(3.0)
    m1 = _bcast(jnp.max(t1, axis=0))
    eq1 = (jnp.where(t1 == m1, one, zero) + jnp.where(t2 == m1, one, zero)
           + jnp.where(t3 == m1, one, zero))
    c1 = _bcast(jnp.sum(eq1, axis=0))
    u1 = jnp.where(t1 < m1, t1, negv)
    u2 = jnp.where(t2 < m1, t2, negv)
    u3 = jnp.where(t3 < m1, t3, negv)
    m2 = _bcast(jnp.max(jnp.maximum(jnp.maximum(u1, u2), u3), axis=0))
    eq2 = (jnp.where(u1 == m2, one, zero) + jnp.where(u2 == m2, one, zero)
           + jnp.where(u3 == m2, one, zero))
    c2 = _bcast(jnp.sum(eq2, axis=0))
    w1 = jnp.where(u1 < m2, u1, negv)
    w2 = jnp.where(u2 < m2, u2, negv)
    w3 = jnp.where(u3 < m2, u3, negv)
    m3 = _bcast(jnp.max(jnp.maximum(jnp.maximum(w1, w2), w3), axis=0))
    kth = jnp.where(c1 >= three, m1, jnp.where(c1 + c2 >= three, m2, m3))

    n1 = jnp.minimum(c1, three)
    n2 = jnp.clip(jnp.minimum(c2, three - n1), 0.0, 3.0)
    n3 = three - n1 - n2
    s_fast = n1 * m1 + n2 * m2 + n3 * m3

    izero = jnp.zeros((L,), jnp.int32)
    ione = jnp.ones((L,), jnp.int32)
    capped = izero
    risk = izero
    for (a1, a2, a3) in tris:
        capped = capped + jnp.where(a1 >= kth, ione, izero)
        capped = capped + jnp.where(a2 >= kth, ione, izero)
        capped = capped + jnp.where(a3 >= kth, ione, izero)
        risk = risk + jnp.where(a3 >= kth, ione, izero)
    n_ge = jnp.sum(capped, axis=0)
    n_risk = jnp.sum(risk, axis=0)
    need_slow = jnp.logical_or(n_ge != 3, n_risk > 0)

    def slow():
        accs0 = (zero,) * ACCS

        @plsc.parallel_loop(0, VECS, ACCS, unroll=4, carry=accs0)
        def accs(i, accs):
            out = []
            for j in range(ACCS):
                x = buf[r, pl.ds((i + j) * L, L)]
                out.append(accs[j] + jnp.where(x >= kth, x, 0.0))
            return tuple(out)

        tot = accs[0]
        for j in range(1, ACCS):
            tot = tot + accs[j]
        return _bcast(jnp.sum(tot, axis=0))

    s = lax.cond(need_slow, slow, lambda: s_fast)
    return kth, 1.0 / s


def _mask_normalize(inb, outb, r, kth, rcp):
    zero = _bcast(0.0)

    @plsc.parallel_loop(0, VECS, 4, unroll=4)
    def _(i):
        for j in range(4):
            sl = pl.ds((i + j) * L, L)
            x = inb[r, sl]
            outb[r, sl] = jnp.where(x >= kth, x * rcp, zero)


def _sc_body(x_hbm, o_hbm, in0, in1, outb, semi0, semi1, semo):
    wid = lax.axis_index("s") * NC + lax.axis_index("c")
    base = wid * ROWS_W
    n_pairs = ROWS_W // CH // 2

    def in_dma(row0, buf, sem):
        return pltpu.make_async_copy(x_hbm.at[pl.ds(row0, CH)], buf, sem)

    def out_dma(row0):
        return pltpu.make_async_copy(outb, o_hbm.at[pl.ds(row0, CH)], semo)

    def do_chunk(c0, inb, t, first_of_pair):
        stats = [_row_stats(inb, r) for r in range(CH)]
        if first_of_pair:
            @pl.when(t > 0)
            def _():
                out_dma(c0).wait()
        else:
            out_dma(c0).wait()
        for r in range(CH):
            kth, rcp = stats[r]
            _mask_normalize(inb, outb, r, kth, rcp)
        out_dma(c0).start()

    in_dma(base, in0, semi0).start()

    def pair(t, _):
        c0 = base + 2 * t * CH
        c1 = c0 + CH
        in_dma(c1, in1, semi1).start()
        in_dma(c0, in0, semi0).wait()
        do_chunk(c0, in0, t, True)

        @pl.when(t < n_pairs - 1)
        def _():
            in_dma(c1 + CH, in0, semi0).start()

        in_dma(c1, in1, semi1).wait()
        do_chunk(c1, in1, t, False)
        return 0

    lax.fori_loop(0, n_pairs, pair, 0)
    out_dma(base).wait()


@jax.jit
def kernel(inputs):
    b, h, n = inputs.shape
    x2d = inputs.reshape(b * h, n)
    mesh = plsc.VectorSubcoreMesh(core_axis_name="c", subcore_axis_name="s")
    k = functools.partial(
        pl.kernel,
        mesh=mesh,
        out_type=jax.ShapeDtypeStruct((b * h, n), jnp.float32),
        scratch_types=[
            pltpu.VMEM((CH, N_COLS), jnp.float32),
            pltpu.VMEM((CH, N_COLS), jnp.float32),
            pltpu.VMEM((CH, N_COLS), jnp.float32),
            pltpu.SemaphoreType.DMA,
            pltpu.SemaphoreType.DMA,
            pltpu.SemaphoreType.DMA,
        ],
        compiler_params=pltpu.CompilerParams(needs_layout_passes=False),
    )(_sc_body)
    return k(x2d).reshape(b, h, n)


# speedup vs baseline: 1.1509x; 1.1509x over previous
import functools

import jax
import jax.numpy as jnp
from jax import lax
from jax.experimental import pallas as pl
from jax.experimental.pallas import tpu as pltpu
from jax.experimental.pallas import tpu_sc as plsc

N_COLS = 8192
N_ROWS = 4096
L = 16
NC, NS = 2, 16
NW = NC * NS
ROWS_W = N_ROWS // NW
CH = 2
VECS = N_COLS // L
ACCS = 4
NEG = -1.0


def _bcast(scalar):
    return jnp.full((L,), scalar, dtype=jnp.float32)


def _tri_update(tri, x):
    t1, t2, t3 = tri
    n1 = jnp.maximum(t1, x)
    lo = jnp.minimum(t1, x)
    n2 = jnp.maximum(t2, lo)
    lo2 = jnp.minimum(t2, lo)
    n3 = jnp.maximum(t3, lo2)
    return n1, n2, n3


def _tri_merge(a, b):
    a1, a2, a3 = a
    b1, b2, b3 = b
    c1 = jnp.maximum(a1, b1)
    c2 = jnp.maximum(jnp.maximum(a2, b2), jnp.minimum(a1, b1))
    c3 = jnp.maximum(
        jnp.maximum(a3, b3),
        jnp.maximum(jnp.minimum(a2, b1), jnp.minimum(a1, b2)),
    )
    return c1, c2, c3


def _row_stats(buf, r):
    neg = _bcast(NEG)
    init = tuple((neg, neg, neg) for _ in range(ACCS))

    @plsc.parallel_loop(0, VECS, ACCS, unroll=4, carry=init)
    def tris(i, tris):
        return tuple(
            _tri_update(tris[j], buf[r, pl.ds((i + j) * L, L)])
            for j in range(ACCS)
        )

    t = tris[0]
    for j in range(1, ACCS):
        t = _tri_merge(t, tris[j])
    t1, t2, t3 = t

    one = _bcast(1.0)
    zero = _bcast(0.0)
    negv = _bcast(NEG)
    three = _bcast(3.0)
    m1 = _bcast(jnp.max(t1, axis=0))
    eq1 = (jnp.where(t1 == m1, one, zero) + jnp.where(t2 == m1, one, zero)
           + jnp.where(t3 == m1, one, zero))
    c1 = _bcast(jnp.sum(eq1, axis=0))
    u1 = jnp.where(t1 < m1, t1, negv)
    u2 = jnp.where(t2 < m1, t2, negv)
    u3 = jnp.where(t3 < m1, t3, negv)
    m2 = _bcast(jnp.max(jnp.maximum(jnp.maximum(u1, u2), u3), axis=0))
    eq2 = (jnp.where(u1 == m2, one, zero) + jnp.where(u2 == m2, one, zero)
           + jnp.where(u3 == m2, one, zero))
    c2 = _bcast(jnp.sum(eq2, axis=0))
    w1 = jnp.where(u1 < m2, u1, negv)
    w2 = jnp.where(u2 < m2, u2, negv)
    w3 = jnp.where(u3 < m2, u3, negv)
    m3 = _bcast(jnp.max(jnp.maximum(jnp.maximum(w1, w2), w3), axis=0))
    kth = jnp.where(c1 >= three, m1, jnp.where(c1 + c2 >= three, m2, m3))

    n1 = jnp.minimum(c1, three)
    n2 = jnp.clip(jnp.minimum(c2, three - n1), 0.0, 3.0)
    n3 = three - n1 - n2
    s_fast = n1 * m1 + n2 * m2 + n3 * m3

    izero = jnp.zeros((L,), jnp.int32)
    ione = jnp.ones((L,), jnp.int32)
    capped = izero
    risk = izero
    for (a1, a2, a3) in tris:
        capped = capped + jnp.where(a1 >= kth, ione, izero)
        capped = capped + jnp.where(a2 >= kth, ione, izero)
        capped = capped + jnp.where(a3 >= kth, ione, izero)
        risk = risk + jnp.where(a3 >= kth, ione, izero)
    n_ge = jnp.sum(capped, axis=0)
    n_risk = jnp.sum(risk, axis=0)
    need_slow = jnp.logical_or(n_ge != 3, n_risk > 0)

    def slow():
        accs0 = (zero,) * ACCS

        @plsc.parallel_loop(0, VECS, ACCS, unroll=4, carry=accs0)
        def accs(i, accs):
            out = []
            for j in range(ACCS):
                x = buf[r, pl.ds((i + j) * L, L)]
                out.append(accs[j] + jnp.where(x >= kth, x, 0.0))
            return tuple(out)

        tot = accs[0]
        for j in range(1, ACCS):
            tot = tot + accs[j]
        return _bcast(jnp.sum(tot, axis=0))

    s = lax.cond(need_slow, slow, lambda: s_fast)
    return kth, 1.0 / s


def _mask_normalize(inb, outb, r, kth, rcp):
    zero = _bcast(0.0)

    @plsc.parallel_loop(0, VECS, 4, unroll=4)
    def _(i):
        for j in range(4):
            sl = pl.ds((i + j) * L, L)
            x = inb[r, sl]
            outb[r, sl] = jnp.where(x >= kth, x * rcp, zero)


def _sc_body(x_hbm, o_hbm, in0, in1, out0, out1, semi0, semi1, semo0, semo1):
    wid = lax.axis_index("s") * NC + lax.axis_index("c")
    base = wid * ROWS_W
    n_pairs = ROWS_W // CH // 2
    outs = (out0, out1)
    semos = (semo0, semo1)

    def in_dma(row0, buf, sem):
        return pltpu.make_async_copy(x_hbm.at[pl.ds(row0, CH)], buf, sem)

    def out_dma(p, row0):
        return pltpu.make_async_copy(outs[p], o_hbm.at[pl.ds(row0, CH)],
                                     semos[p])

    def do_chunk(c0, inb, t, p):
        stats = [_row_stats(inb, r) for r in range(CH)]

        @pl.when(t > 0)
        def _():
            out_dma(p, c0).wait()

        for r in range(CH):
            kth, rcp = stats[r]
            _mask_normalize(inb, outs[p], r, kth, rcp)
        out_dma(p, c0).start()

    in_dma(base, in0, semi0).start()

    def pair(t, _):
        c0 = base + 2 * t * CH
        c1 = c0 + CH
        in_dma(c1, in1, semi1).start()
        in_dma(c0, in0, semi0).wait()
        do_chunk(c0, in0, t, 0)

        @pl.when(t < n_pairs - 1)
        def _():
            in_dma(c1 + CH, in0, semi0).start()

        in_dma(c1, in1, semi1).wait()
        do_chunk(c1, in1, t, 1)
        return 0

    lax.fori_loop(0, n_pairs, pair, 0)
    out_dma(0, base).wait()
    out_dma(1, base).wait()


@jax.jit
def kernel(inputs):
    b, h, n = inputs.shape
    x2d = inputs.reshape(b * h, n)
    mesh = plsc.VectorSubcoreMesh(core_axis_name="c", subcore_axis_name="s")
    k = functools.partial(
        pl.kernel,
        mesh=mesh,
        out_type=jax.ShapeDtypeStruct((b * h, n), jnp.float32),
        scratch_types=[
            pltpu.VMEM((CH, N_COLS), jnp.float32),
            pltpu.VMEM((CH, N_COLS), jnp.float32),
            pltpu.VMEM((CH, N_COLS), jnp.float32),
            pltpu.VMEM((CH, N_COLS), jnp.float32),
            pltpu.SemaphoreType.DMA,
            pltpu.SemaphoreType.DMA,
            pltpu.SemaphoreType.DMA,
            pltpu.SemaphoreType.DMA,
        ],
        compiler_params=pltpu.CompilerParams(needs_layout_passes=False),
    )(_sc_body)
    return k(x2d).reshape(b, h, n)


# speedup vs baseline: 1.1559x; 1.0043x over previous
import functools

import jax
import jax.numpy as jnp
from jax import lax
from jax.experimental import pallas as pl
from jax.experimental.pallas import tpu as pltpu
from jax.experimental.pallas import tpu_sc as plsc

N_COLS = 8192
N_ROWS = 4096
L = 16
NC, NS = 2, 16
NW = NC * NS
ROWS_W = N_ROWS // NW
CH = 2
VECS = N_COLS // L
ACCS = 4
NEG = -1.0


def _bcast(scalar):
    return jnp.full((L,), scalar, dtype=jnp.float32)


def _tri_update(tri, x):
    t1, t2, t3 = tri
    n1 = jnp.maximum(t1, x)
    lo = jnp.minimum(t1, x)
    n2 = jnp.maximum(t2, lo)
    lo2 = jnp.minimum(t2, lo)
    n3 = jnp.maximum(t3, lo2)
    return n1, n2, n3


def _tri_merge(a, b):
    a1, a2, a3 = a
    b1, b2, b3 = b
    c1 = jnp.maximum(a1, b1)
    c2 = jnp.maximum(jnp.maximum(a2, b2), jnp.minimum(a1, b1))
    c3 = jnp.maximum(
        jnp.maximum(a3, b3),
        jnp.maximum(jnp.minimum(a2, b1), jnp.minimum(a1, b2)),
    )
    return c1, c2, c3


def _row_stats(buf, r):
    neg = _bcast(NEG)
    init = tuple((neg, neg, neg) for _ in range(ACCS))

    @plsc.parallel_loop(0, VECS, ACCS, unroll=4, carry=init)
    def tris(i, tris):
        return tuple(
            _tri_update(tris[j], buf[r, pl.ds((i + j) * L, L)])
            for j in range(ACCS)
        )

    t = tris[0]
    for j in range(1, ACCS):
        t = _tri_merge(t, tris[j])
    t1, t2, t3 = t

    one = _bcast(1.0)
    zero = _bcast(0.0)
    negv = _bcast(NEG)
    three = _bcast(3.0)
    m1 = _bcast(jnp.max(t1, axis=0))
    eq1 = (jnp.where(t1 == m1, one, zero) + jnp.where(t2 == m1, one, zero)
           + jnp.where(t3 == m1, one, zero))
    c1 = _bcast(jnp.sum(eq1, axis=0))
    u1 = jnp.where(t1 < m1, t1, negv)
    u2 = jnp.where(t2 < m1, t2, negv)
    u3 = jnp.where(t3 < m1, t3, negv)
    m2 = _bcast(jnp.max(jnp.maximum(jnp.maximum(u1, u2), u3), axis=0))
    eq2 = (jnp.where(u1 == m2, one, zero) + jnp.where(u2 == m2, one, zero)
           + jnp.where(u3 == m2, one, zero))
    c2 = _bcast(jnp.sum(eq2, axis=0))
    w1 = jnp.where(u1 < m2, u1, negv)
    w2 = jnp.where(u2 < m2, u2, negv)
    w3 = jnp.where(u3 < m2, u3, negv)
    m3 = _bcast(jnp.max(jnp.maximum(jnp.maximum(w1, w2), w3), axis=0))
    kth = jnp.where(c1 >= three, m1, jnp.where(c1 + c2 >= three, m2, m3))

    n1 = jnp.minimum(c1, three)
    n2 = jnp.clip(jnp.minimum(c2, three - n1), 0.0, 3.0)
    n3 = three - n1 - n2
    s_fast = n1 * m1 + n2 * m2 + n3 * m3

    izero = jnp.zeros((L,), jnp.int32)
    ione = jnp.ones((L,), jnp.int32)
    capped = izero
    risk = izero
    for (a1, a2, a3) in tris:
        capped = capped + jnp.where(a1 >= kth, ione, izero)
        capped = capped + jnp.where(a2 >= kth, ione, izero)
        capped = capped + jnp.where(a3 >= kth, ione, izero)
        risk = risk + jnp.where(a3 >= kth, ione, izero)
    n_ge = jnp.sum(capped, axis=0)
    n_risk = jnp.sum(risk, axis=0)
    need_slow = jnp.logical_or(n_ge != 3, n_risk > 0)

    def slow():
        accs0 = (zero,) * ACCS

        @plsc.parallel_loop(0, VECS, ACCS, unroll=4, carry=accs0)
        def accs(i, accs):
            out = []
            for j in range(ACCS):
                x = buf[r, pl.ds((i + j) * L, L)]
                out.append(accs[j] + jnp.where(x >= kth, x, 0.0))
            return tuple(out)

        tot = accs[0]
        for j in range(1, ACCS):
            tot = tot + accs[j]
        return _bcast(jnp.sum(tot, axis=0))

    s = lax.cond(need_slow, slow, lambda: s_fast)
    return kth, 1.0 / s


def _mask_normalize(inb, outb, r, kth, rcp):
    zero = _bcast(0.0)

    @plsc.parallel_loop(0, VECS, 4, unroll=4)
    def _(i):
        for j in range(4):
            sl = pl.ds((i + j) * L, L)
            x = inb[r, sl]
            outb[r, sl] = jnp.where(x >= kth, x * rcp, zero)


def _sc_body(x_hbm, o_hbm, in0, in1, outb, semi0, semi1, semo):
    wid = lax.axis_index("s") * NC + lax.axis_index("c")
    base = wid * ROWS_W
    n_pairs = ROWS_W // CH // 2

    def in_dma(row0, buf, sem):
        return pltpu.make_async_copy(x_hbm.at[pl.ds(row0, CH)], buf, sem)

    def out_dma(row0):
        return pltpu.make_async_copy(outb, o_hbm.at[pl.ds(row0, CH)], semo)

    def do_chunk(c0, inb, t, first_of_pair):
        stats = [_row_stats(inb, r) for r in range(CH)]
        if first_of_pair:
            @pl.when(t > 0)
            def _():
                out_dma(c0).wait()
        else:
            out_dma(c0).wait()
        for r in range(CH):
            kth, rcp = stats[r]
            _mask_normalize(inb, outb, r, kth, rcp)
        out_dma(c0).start()

    in_dma(base, in0, semi0).start()

    def pair(t, _):
        c0 = base + 2 * t * CH
        c1 = c0 + CH
        in_dma(c1, in1, semi1).start()
        in_dma(c0, in0, semi0).wait()
        do_chunk(c0, in0, t, True)

        @pl.when(t < n_pairs - 1)
        def _():
            in_dma(c1 + CH, in0, semi0).start()

        in_dma(c1, in1, semi1).wait()
        do_chunk(c1, in1, t, False)
        return 0

    lax.fori_loop(0, n_pairs, pair, 0)
    out_dma(base).wait()


@jax.jit
def kernel(inputs):
    b, h, n = inputs.shape
    x2d = inputs.reshape(b * h, n)
    mesh = plsc.VectorSubcoreMesh(core_axis_name="c", subcore_axis_name="s")
    k = functools.partial(
        pl.kernel,
        mesh=mesh,
        out_type=jax.ShapeDtypeStruct((b * h, n), jnp.float32),
        scratch_types=[
            pltpu.VMEM((CH, N_COLS), jnp.float32),
            pltpu.VMEM((CH, N_COLS), jnp.float32),
            pltpu.VMEM((CH, N_COLS), jnp.float32),
            pltpu.SemaphoreType.DMA,
            pltpu.SemaphoreType.DMA,
            pltpu.SemaphoreType.DMA,
        ],
        compiler_params=pltpu.CompilerParams(needs_layout_passes=False),
    )(_sc_body)
    return k(x2d).reshape(b, h, n)
